# baseline (speedup 1.0000x reference)
import functools

import jax
import jax.numpy as jnp
from jax import lax
from jax.experimental import pallas as pl
from jax.experimental.pallas import tpu as pltpu
from jax.experimental.pallas import tpu_sc as plsc

NC = 2
NS = 16
NW = NC * NS
GC = 96
LANES = 16
_FRAC0 = 120.0 / 210.0


def _mm_tanh_body(x_ref, w_ref, o_ref):
    o_ref[...] = jnp.tanh(
        lax.dot_general(x_ref[...], w_ref[...], (((1,), (0,)), ((), ())),
                        precision=lax.Precision.HIGHEST,
                        preferred_element_type=jnp.float32))


def _combine_body(a_ref, b_ref, o_ref):
    o_ref[...] = a_ref[...] + b_ref[...]


def _make_sc_agg(N, N_pad, D, C0, C1):
    mesh = plsc.VectorSubcoreMesh(core_axis_name="c", subcore_axis_name="s",
                                  num_cores=NC, num_subcores=NS)
    rows_per_tile = N_pad // NS
    Cmax = max(C0, C1)
    assert C0 % 3 == 0 and C1 % 3 == 0 and rows_per_tile % GC == 0

    @functools.partial(
        pl.kernel,
        out_type=(jax.ShapeDtypeStruct((N_pad, D), jnp.float32),
                  jax.ShapeDtypeStruct((N_pad, D), jnp.float32)),
        mesh=mesh,
        compiler_params=pltpu.CompilerParams(needs_layout_passes=False),
        scratch_types=[
            pltpu.VMEM((9, 128), jnp.int32),
            pltpu.VMEM((GC, D), jnp.float32),
            pltpu.VMEM((GC, D), jnp.float32),
            pltpu.VMEM((GC, D), jnp.float32),
            pltpu.VMEM_SHARED((N_pad, D), jnp.float32),
            pltpu.SemaphoreType.DMA,
            pltpu.SemaphoreType.DMA,
            pltpu.SemaphoreType.DMA,
            pltpu.SemaphoreType.DMA,
            pltpu.SemaphoreType.DMA,
            pltpu.SemaphoreType.DMA,
            pltpu.SemaphoreType.DMA,
            pltpu.SemaphoreType.DMA,
            pltpu.SemaphoreType.DMA,
        ],
    )
    def sc_agg(h_hbm, ed_hbm, p0_hbm, p1_hbm,
               ring, b0, b1, b2, acc_sh,
               es0, es1, es2, gs0, gs1, gs2, ss0, ss1, ss2):
        cid = lax.axis_index("c")
        sid = lax.axis_index("s")
        wid = sid * NC + cid
        bufs = (b0, b1, b2)
        ess = (es0, es1, es2)
        gss = (gs0, gs1, gs2)
        sss = (ss0, ss1, ss2)
        Cv = jnp.where(cid == 0, C0, C1)
        Cm1 = Cv - 1

        def _zrow(r, carry):
            for j in range(D // LANES):
                b0[r, pl.ds(j * LANES, LANES)] = jnp.zeros((LANES,),
                                                           jnp.float32)
            return carry
        lax.fori_loop(0, GC, _zrow, 0)
        base = sid * rows_per_tile
        for k in range(rows_per_tile // GC):
            pltpu.sync_copy(b0, acc_sh.at[pl.ds(base + k * GC, GC)])
        plsc.subcore_barrier()

        def _refill(row, sl, sem):
            pltpu.async_copy(ed_hbm.at[wid, row],
                             ring.at[pl.ds(3 * sl, 3)], sem)

        def _ewait(sem):
            pltpu.make_async_copy(ed_hbm.at[wid, 0],
                                  ring.at[pl.ds(0, 3)], sem).wait()

        def _gather(sl, buf, sem):
            pltpu.async_copy(h_hbm.at[ring.at[3 * sl, pl.ds(0, GC)]],
                             buf, sem)

        def _gwait(buf, sem):
            pltpu.make_async_copy(h_hbm.at[ring.at[0, pl.ds(0, GC)]],
                                  buf, sem).wait()

        zidx = jnp.zeros((LANES,), jnp.int32)

        def _scatter(buf, sl, sem):
            for k in range(GC // LANES):
                dv = ring[3 * sl + 1, pl.ds(k * LANES, LANES)]
                pltpu.async_copy(buf.at[pl.ds(k * LANES, LANES)],
                                 acc_sh.at[dv], sem, add=True)

        def _sdrain(buf, sem):
            for k in range(GC // LANES):
                pltpu.make_async_copy(buf.at[pl.ds(k * LANES, LANES)],
                                      acc_sh.at[zidx], sem).wait()

        def _scale(buf, sl):
            wr = jnp.full((LANES,), 3 * sl + 2, jnp.int32)

            @plsc.parallel_loop(0, GC, 1, unroll=4)
            def _edge(e):
                wbits = plsc.load_gather(
                    ring, [wr, jnp.full((LANES,), e, jnp.int32)])
                ws = plsc.bitcast(wbits, jnp.float32)
                for j in range(D // LANES):
                    fs = pl.ds(j * LANES, LANES)
                    buf[e, fs] = buf[e, fs] * ws

        _refill(0, 0, es0)
        _refill(1, 1, es1)
        _refill(2, 2, es2)
        _ewait(es0)
        _gather(0, b0, gs0)

        def _body(i, carry):
            for k in range(3):
                r = 3 * i + k
                kp1 = (k + 1) % 3

                @pl.when(r >= 2)
                def _():
                    _sdrain(bufs[kp1], sss[kp1])
                _ewait(ess[kp1])
                _gather(kp1, bufs[kp1], gss[kp1])
                _gwait(bufs[k], gss[k])
                _scale(bufs[k], k)
                _scatter(bufs[k], k, sss[k])
                _refill(jnp.minimum(r + 3, Cm1), k, ess[k])
            return carry
        lax.fori_loop(0, Cv // 3, _body, 0)

        _sdrain(bufs[1], sss[1])
        _sdrain(bufs[2], sss[2])
        _gwait(bufs[0], gss[0])
        _ewait(ess[1])
        _ewait(ess[2])
        plsc.subcore_barrier()

        @pl.when(cid == 0)
        def _():
            pltpu.sync_copy(acc_sh.at[pl.ds(base, rows_per_tile)],
                            p0_hbm.at[pl.ds(base, rows_per_tile)])

        @pl.when(cid == 1)
        def _():
            pltpu.sync_copy(acc_sh.at[pl.ds(base, rows_per_tile)],
                            p1_hbm.at[pl.ds(base, rows_per_tile)])

    return sc_agg


def kernel(inputs, edge_index, edge_weight, W, b):
    N, D = inputs.shape
    E = edge_weight.shape[0]

    BM = 2000
    h = pl.pallas_call(
        _mm_tanh_body,
        grid=(N // BM,),
        in_specs=[pl.BlockSpec((BM, D), lambda i: (i, 0)),
                  pl.BlockSpec((D, D), lambda i: (0, 0))],
        out_specs=pl.BlockSpec((BM, D), lambda i: (i, 0)),
        out_shape=jax.ShapeDtypeStruct((N, D), jnp.float32),
    )(inputs, W)

    csum = -(-E // (NS * GC))
    C0 = max(3, int(round(_FRAC0 * csum / 3.0)) * 3)
    C1 = max(3, -(-(csum - C0) // 3) * 3)
    Cmax = max(C0, C1)
    E_cap = NS * (C0 + C1) * GC
    pad = E_cap - E
    n0 = NS * C0 * GC

    def _slab(x):
        x = jnp.concatenate([x, jnp.zeros((pad,), jnp.int32)])
        a0 = jnp.pad(x[:n0].reshape(NS, C0, GC),
                     ((0, 0), (0, Cmax - C0), (0, 0)))
        a1 = jnp.pad(x[n0:].reshape(NS, C1, GC),
                     ((0, 0), (0, Cmax - C1), (0, 0)))
        a = jnp.stack([a0, a1], axis=1).reshape(NW, Cmax, GC)
        return jnp.pad(a, ((0, 0), (0, 0), (0, 128 - GC)))

    src = _slab(edge_index[0])
    dst = _slab(edge_index[1])
    wbits = _slab(lax.bitcast_convert_type(edge_weight, jnp.int32))
    edata = jnp.stack([src, dst, wbits], axis=2)

    stripe = NS * GC
    N_pad = ((N + stripe - 1) // stripe) * stripe

    p0, p1 = _make_sc_agg(N, N_pad, D, C0, C1)(h, edata)

    out = pl.pallas_call(
        _combine_body,
        grid=(N // BM,),
        in_specs=[pl.BlockSpec((BM, D), lambda i: (i, 0)),
                  pl.BlockSpec((BM, D), lambda i: (i, 0))],
        out_specs=pl.BlockSpec((BM, D), lambda i: (i, 0)),
        out_shape=jax.ShapeDtypeStruct((N, D), jnp.float32),
    )(p0, p1)
    return out


# speedup vs baseline: 1.0332x; 1.0332x over previous
import functools

import jax
import jax.numpy as jnp
from jax import lax
from jax.experimental import pallas as pl
from jax.experimental.pallas import tpu as pltpu
from jax.experimental.pallas import tpu_sc as plsc

NC = 2
NS = 16
NW = NC * NS
GC = 96
LANES = 16
_FRAC0 = 135.0 / 210.0


def _mm_tanh_body(x_ref, w_ref, o_ref):
    o_ref[...] = jnp.tanh(
        lax.dot_general(x_ref[...], w_ref[...], (((1,), (0,)), ((), ())),
                        precision=lax.Precision.HIGHEST,
                        preferred_element_type=jnp.float32))


def _combine_body(a_ref, b_ref, o_ref):
    o_ref[...] = a_ref[...] + b_ref[...]


def _make_sc_agg(N, N_pad, D, C0, C1):
    mesh = plsc.VectorSubcoreMesh(core_axis_name="c", subcore_axis_name="s",
                                  num_cores=NC, num_subcores=NS)
    rows_per_tile = N_pad // NS
    Cmax = max(C0, C1)
    assert C0 % 3 == 0 and C1 % 3 == 0 and rows_per_tile % GC == 0

    @functools.partial(
        pl.kernel,
        out_type=(jax.ShapeDtypeStruct((N_pad, D), jnp.float32),
                  jax.ShapeDtypeStruct((N_pad, D), jnp.float32)),
        mesh=mesh,
        compiler_params=pltpu.CompilerParams(needs_layout_passes=False),
        scratch_types=[
            pltpu.VMEM((9, 128), jnp.int32),
            pltpu.VMEM((GC, D), jnp.float32),
            pltpu.VMEM((GC, D), jnp.float32),
            pltpu.VMEM((GC, D), jnp.float32),
            pltpu.VMEM_SHARED((N_pad, D), jnp.float32),
            pltpu.SemaphoreType.DMA,
            pltpu.SemaphoreType.DMA,
            pltpu.SemaphoreType.DMA,
            pltpu.SemaphoreType.DMA,
            pltpu.SemaphoreType.DMA,
            pltpu.SemaphoreType.DMA,
            pltpu.SemaphoreType.DMA,
            pltpu.SemaphoreType.DMA,
            pltpu.SemaphoreType.DMA,
        ],
    )
    def sc_agg(h_hbm, ed_hbm, p0_hbm, p1_hbm,
               ring, b0, b1, b2, acc_sh,
               es0, es1, es2, gs0, gs1, gs2, ss0, ss1, ss2):
        cid = lax.axis_index("c")
        sid = lax.axis_index("s")
        wid = sid * NC + cid
        bufs = (b0, b1, b2)
        ess = (es0, es1, es2)
        gss = (gs0, gs1, gs2)
        sss = (ss0, ss1, ss2)
        Cv = jnp.where(cid == 0, C0, C1)
        Cm1 = Cv - 1

        def _zrow(r, carry):
            for j in range(D // LANES):
                b0[r, pl.ds(j * LANES, LANES)] = jnp.zeros((LANES,),
                                                           jnp.float32)
            return carry
        lax.fori_loop(0, GC, _zrow, 0)
        base = sid * rows_per_tile
        for k in range(rows_per_tile // GC):
            pltpu.sync_copy(b0, acc_sh.at[pl.ds(base + k * GC, GC)])
        plsc.subcore_barrier()

        def _refill(row, sl, sem):
            pltpu.async_copy(ed_hbm.at[wid, row],
                             ring.at[pl.ds(3 * sl, 3)], sem)

        def _ewait(sem):
            pltpu.make_async_copy(ed_hbm.at[wid, 0],
                                  ring.at[pl.ds(0, 3)], sem).wait()

        def _gather(sl, buf, sem):
            pltpu.async_copy(h_hbm.at[ring.at[3 * sl, pl.ds(0, GC)]],
                             buf, sem)

        def _gwait(buf, sem):
            pltpu.make_async_copy(h_hbm.at[ring.at[0, pl.ds(0, GC)]],
                                  buf, sem).wait()

        zidx = jnp.zeros((LANES,), jnp.int32)

        def _scatter(buf, sl, sem):
            for k in range(GC // LANES):
                dv = ring[3 * sl + 1, pl.ds(k * LANES, LANES)]
                pltpu.async_copy(buf.at[pl.ds(k * LANES, LANES)],
                                 acc_sh.at[dv], sem, add=True)

        def _sdrain(buf, sem):
            for k in range(GC // LANES):
                pltpu.make_async_copy(buf.at[pl.ds(k * LANES, LANES)],
                                      acc_sh.at[zidx], sem).wait()

        def _scale(buf, sl):
            wr = jnp.full((LANES,), 3 * sl + 2, jnp.int32)

            @plsc.parallel_loop(0, GC, 1, unroll=4)
            def _edge(e):
                wbits = plsc.load_gather(
                    ring, [wr, jnp.full((LANES,), e, jnp.int32)])
                ws = plsc.bitcast(wbits, jnp.float32)
                for j in range(D // LANES):
                    fs = pl.ds(j * LANES, LANES)
                    buf[e, fs] = buf[e, fs] * ws

        _refill(0, 0, es0)
        _refill(1, 1, es1)
        _refill(2, 2, es2)
        _ewait(es0)
        _gather(0, b0, gs0)

        def _body(i, carry):
            for k in range(3):
                r = 3 * i + k
                kp1 = (k + 1) % 3

                @pl.when(r >= 2)
                def _():
                    _sdrain(bufs[kp1], sss[kp1])
                _ewait(ess[kp1])
                _gather(kp1, bufs[kp1], gss[kp1])
                _gwait(bufs[k], gss[k])
                _scale(bufs[k], k)
                _scatter(bufs[k], k, sss[k])
                _refill(jnp.minimum(r + 3, Cm1), k, ess[k])
            return carry
        lax.fori_loop(0, Cv // 3, _body, 0)

        _sdrain(bufs[1], sss[1])
        _sdrain(bufs[2], sss[2])
        _gwait(bufs[0], gss[0])
        _ewait(ess[1])
        _ewait(ess[2])
        plsc.subcore_barrier()

        @pl.when(cid == 0)
        def _():
            pltpu.sync_copy(acc_sh.at[pl.ds(base, rows_per_tile)],
                            p0_hbm.at[pl.ds(base, rows_per_tile)])

        @pl.when(cid == 1)
        def _():
            pltpu.sync_copy(acc_sh.at[pl.ds(base, rows_per_tile)],
                            p1_hbm.at[pl.ds(base, rows_per_tile)])

    return sc_agg


def kernel(inputs, edge_index, edge_weight, W, b):
    N, D = inputs.shape
    E = edge_weight.shape[0]

    BM = 2000
    h = pl.pallas_call(
        _mm_tanh_body,
        grid=(N // BM,),
        in_specs=[pl.BlockSpec((BM, D), lambda i: (i, 0)),
                  pl.BlockSpec((D, D), lambda i: (0, 0))],
        out_specs=pl.BlockSpec((BM, D), lambda i: (i, 0)),
        out_shape=jax.ShapeDtypeStruct((N, D), jnp.float32),
    )(inputs, W)

    csum = -(-E // (NS * GC))
    C0 = max(3, int(round(_FRAC0 * csum / 3.0)) * 3)
    C1 = max(3, -(-(csum - C0) // 3) * 3)
    Cmax = max(C0, C1)
    E_cap = NS * (C0 + C1) * GC
    pad = E_cap - E
    n0 = NS * C0 * GC

    def _slab(x):
        x = jnp.concatenate([x, jnp.zeros((pad,), jnp.int32)])
        a0 = jnp.pad(x[:n0].reshape(NS, C0, GC),
                     ((0, 0), (0, Cmax - C0), (0, 0)))
        a1 = jnp.pad(x[n0:].reshape(NS, C1, GC),
                     ((0, 0), (0, Cmax - C1), (0, 0)))
        a = jnp.stack([a0, a1], axis=1).reshape(NW, Cmax, GC)
        return jnp.pad(a, ((0, 0), (0, 0), (0, 128 - GC)))

    src = _slab(edge_index[0])
    dst = _slab(edge_index[1])
    wbits = _slab(lax.bitcast_convert_type(edge_weight, jnp.int32))
    edata = jnp.stack([src, dst, wbits], axis=2)

    stripe = NS * GC
    N_pad = ((N + stripe - 1) // stripe) * stripe

    p0, p1 = _make_sc_agg(N, N_pad, D, C0, C1)(h, edata)

    out = pl.pallas_call(
        _combine_body,
        grid=(N // BM,),
        in_specs=[pl.BlockSpec((BM, D), lambda i: (i, 0)),
                  pl.BlockSpec((BM, D), lambda i: (i, 0))],
        out_specs=pl.BlockSpec((BM, D), lambda i: (i, 0)),
        out_shape=jax.ShapeDtypeStruct((N, D), jnp.float32),
    )(p0, p1)
    return out
